# baseline (speedup 1.0000x reference)
import functools

import jax
import jax.numpy as jnp
from jax import lax
from jax.experimental import pallas as pl
from jax.experimental.pallas import tpu as pltpu
from jax.experimental.pallas import tpu_sc as plsc

N_NODES = 10000
N_EDGES = 320000
D_LAT = 128
D_EDGE_IN = 16

NC, NS = 1, 16
NW = NC * NS
EPW = N_EDGES // NW
CHUNK = 125
NCHUNK = EPW // CHUNK
IB = 20
NB = NCHUNK // IB
W_STEP = 624
W_ROWS = 640

_HIGH = jax.lax.Precision.HIGHEST


def _dot(a, b):
    return jnp.dot(a, b, precision=_HIGH, preferred_element_type=jnp.float32)


def _acc_window(ref, s):
    return ref.at[pl.ds(s * W_STEP, W_ROWS)]


def _spmm_phase(t_hbm, src4d_hbm, dst4d_hbm,
                src_v, dst_v, rows_a, rows_b, acc, sem_a, sem_b, wid):
    def outer(b, carry):
        pltpu.sync_copy(src4d_hbm.at[wid, b], src_v)
        pltpu.sync_copy(dst4d_hbm.at[wid, b], dst_v)
        pltpu.async_copy(t_hbm.at[src_v.at[0]], rows_a, sem_a)

        def body(jj, carry2):
            @pl.when(jj % 2 == 0)
            def _even():
                @pl.when(jj + 1 < IB)
                def _():
                    pltpu.async_copy(t_hbm.at[src_v.at[jj + 1]], rows_b, sem_b)
                pltpu.make_async_copy(t_hbm.at[src_v.at[jj]], rows_a,
                                      sem_a).wait()
                pltpu.sync_copy(rows_a, acc.at[dst_v.at[jj]], add=True)

            @pl.when(jj % 2 == 1)
            def _odd():
                @pl.when(jj + 1 < IB)
                def _():
                    pltpu.async_copy(t_hbm.at[src_v.at[jj + 1]], rows_a, sem_a)
                pltpu.make_async_copy(t_hbm.at[src_v.at[jj]], rows_b,
                                      sem_b).wait()
                pltpu.sync_copy(rows_b, acc.at[dst_v.at[jj]], add=True)

            return carry2

        return lax.fori_loop(0, IB, body, carry)

    lax.fori_loop(0, NB, outer, 0)


def _segsum_phase(xe_hbm, dst4d_hbm, dst_v, rows_a, rows_b, acc,
                  sem_a, sem_b, wid):
    def outer(b, carry):
        pltpu.sync_copy(dst4d_hbm.at[wid, b], dst_v)
        pltpu.async_copy(xe_hbm.at[wid * NCHUNK + b * IB], rows_a, sem_a)

        def body(jj, carry2):
            j = b * IB + jj

            @pl.when(jj % 2 == 0)
            def _even():
                @pl.when(jj + 1 < IB)
                def _():
                    pltpu.async_copy(xe_hbm.at[wid * NCHUNK + j + 1], rows_b,
                                     sem_b)
                pltpu.make_async_copy(xe_hbm.at[wid * NCHUNK + j], rows_a,
                                      sem_a).wait()
                pltpu.sync_copy(rows_a, acc.at[dst_v.at[jj]], add=True)

            @pl.when(jj % 2 == 1)
            def _odd():
                @pl.when(jj + 1 < IB)
                def _():
                    pltpu.async_copy(xe_hbm.at[wid * NCHUNK + j + 1], rows_a,
                                     sem_a)
                pltpu.make_async_copy(xe_hbm.at[wid * NCHUNK + j], rows_b,
                                      sem_b).wait()
                pltpu.sync_copy(rows_b, acc.at[dst_v.at[jj]], add=True)

            return carry2

        return lax.fori_loop(0, IB, body, carry)

    lax.fori_loop(0, NB, outer, 0)


def _sc_fused_body(xe_hbm, t_hbm, src4d_hbm, dst4d_hbm, z_hbm,
                   sxd_out, msum_out,
                   src_v, dst_v, rows_a, rows_b, acc, sem_a, sem_b):
    c = lax.axis_index("c")
    s = lax.axis_index("s")
    wid = s * NC + c

    pltpu.sync_copy(z_hbm, _acc_window(acc, s))
    plsc.subcore_barrier()
    _segsum_phase(xe_hbm, dst4d_hbm, dst_v, rows_a, rows_b, acc,
                  sem_a, sem_b, wid)
    plsc.subcore_barrier()
    pltpu.sync_copy(_acc_window(acc, s), _acc_window(sxd_out.at[c], s))
    plsc.subcore_barrier()
    pltpu.sync_copy(z_hbm, _acc_window(acc, s))
    plsc.subcore_barrier()
    _spmm_phase(t_hbm, src4d_hbm, dst4d_hbm,
                src_v, dst_v, rows_a, rows_b, acc, sem_a, sem_b, wid)
    plsc.subcore_barrier()
    pltpu.sync_copy(_acc_window(acc, s), _acc_window(msum_out.at[c], s))


def _sc_spmm_body(t_hbm, src4d_hbm, dst4d_hbm, z_hbm,
                  msum_out,
                  src_v, dst_v, rows_a, rows_b, acc, sem_a, sem_b):
    c = lax.axis_index("c")
    s = lax.axis_index("s")
    wid = s * NC + c

    pltpu.sync_copy(z_hbm, _acc_window(acc, s))
    plsc.subcore_barrier()
    _spmm_phase(t_hbm, src4d_hbm, dst4d_hbm,
                src_v, dst_v, rows_a, rows_b, acc, sem_a, sem_b, wid)
    plsc.subcore_barrier()
    pltpu.sync_copy(_acc_window(acc, s), _acc_window(msum_out.at[c], s))


_SC_SCRATCH = [
    pltpu.VMEM((IB, CHUNK), jnp.int32),
    pltpu.VMEM((IB, CHUNK), jnp.int32),
    pltpu.VMEM((CHUNK, D_LAT), jnp.float32),
    pltpu.VMEM((CHUNK, D_LAT), jnp.float32),
    pltpu.VMEM_SHARED((N_NODES, D_LAT), jnp.float32),
    pltpu.SemaphoreType.DMA,
    pltpu.SemaphoreType.DMA,
]
_SC_MESH = dict(core_axis_name="c", subcore_axis_name="s",
                num_cores=NC, num_subcores=NS)
_OUT128 = jax.ShapeDtypeStruct((NC, N_NODES, D_LAT), jnp.float32)


@functools.cache
def _sc_fused():
    return pl.kernel(
        _sc_fused_body,
        out_type=(_OUT128, _OUT128),
        mesh=plsc.VectorSubcoreMesh(**_SC_MESH),
        scratch_types=list(_SC_SCRATCH),
    )


@functools.cache
def _sc_spmm():
    return pl.kernel(
        _sc_spmm_body,
        out_type=_OUT128,
        mesh=plsc.VectorSubcoreMesh(**_SC_MESH),
        scratch_types=list(_SC_SCRATCH),
    )


def _tc_prep_body(wn1, bn1, wn2, bn2, we1, be1, we2, be2,
                  ew1_0, eb1_0, ew2_0, eb2_0, nw1_0, nb1_0, nw2_0, nb2_0,
                  ew1_1, eb1_1, ew2_1, eb2_1, nw1_1, nb1_1, nw2_1, nb2_1,
                  dw1, db1, dw2, db2,
                  en_o, bn_o, ee_o, be_o,
                  ia0_o, b0_o, c0_o, de0_o, ip0_o, q0_o, cn0_o,
                  ia1_o, b1_o, c1_o, de1_o, ip1_o, q1_o, cn1_o,
                  d_o, dd_o):
    eye = jnp.eye(D_LAT, dtype=jnp.float32)
    en_o[...] = _dot(wn1[...], wn2[...])
    bn_o[...] = _dot(bn1[...], wn2[...]) + bn2[...]
    ee_o[...] = _dot(we1[...], we2[...])
    be_o[...] = _dot(be1[...], we2[...]) + be2[...]

    for (ew1, eb1, ew2, eb2, nw1, nb1, nw2, nb2,
         ia_o, b_o, c_o, de_o, ip_o, q_o, cn_o) in (
            (ew1_0, eb1_0, ew2_0, eb2_0, nw1_0, nb1_0, nw2_0, nb2_0,
             ia0_o, b0_o, c0_o, de0_o, ip0_o, q0_o, cn0_o),
            (ew1_1, eb1_1, ew2_1, eb2_1, nw1_1, nb1_1, nw2_1, nb2_1,
             ia1_o, b1_o, c1_o, de1_o, ip1_o, q1_o, cn1_o)):
        w1 = ew1[...]
        w2 = ew2[...]
        ia_o[...] = eye + _dot(w1[:D_LAT], w2)
        b_o[...] = _dot(w1[D_LAT:2 * D_LAT], w2)
        c_o[...] = _dot(w1[2 * D_LAT:], w2)
        de_o[...] = _dot(eb1[...], w2) + eb2[...]
        m1 = nw1[...]
        m2 = nw2[...]
        ip_o[...] = eye + _dot(m1[:D_LAT], m2)
        q_o[...] = _dot(m1[D_LAT:], m2)
        cn_o[...] = _dot(nb1[...], m2) + nb2[...]

    d_o[...] = _dot(dw1[...], dw2[...])
    dd_o[...] = _dot(db1[...], dw2[...]) + db2[...]


def _tc_encode_body(xn, en, bn, b0, c0, v0_o, t1_o, u1_o):
    v0 = _dot(xn[...], en[...]) + bn[...]
    v0_o[...] = v0
    t1_o[...] = _dot(v0, b0[...])
    u1_o[...] = _dot(v0, c0[...])


def _tc_step1_body(sxd_p, msum_p, v0, u1,
                   ee, be, ia0, de0, ip0, q0, cn0, b1, c1,
                   agg1_o, v1_o, t2_o, u2_o, deg_o):
    sxd = jnp.sum(sxd_p[...], axis=0)
    msum_a = msum_p[...]
    sx = sxd[:, :D_EDGE_IN]
    deg = sxd[:, D_EDGE_IN:D_EDGE_IN + 1]
    deg_o[...] = jnp.broadcast_to(deg, deg_o.shape)
    eagg0 = _dot(sx, ee[...]) + deg * be[...]
    agg1 = (_dot(eagg0, ia0[...]) + jnp.sum(msum_a, axis=0)
            + deg * u1[...] + deg * de0[...])
    v1 = _dot(v0[...], ip0[...]) + _dot(agg1, q0[...]) + cn0[...]
    agg1_o[...] = agg1
    v1_o[...] = v1
    t2_o[...] = _dot(v1, b1[...])
    u2_o[...] = _dot(v1, c1[...])


def _tc_step2_body(degb, agg1, msum_p, v1, u2,
                   ia1, de1, ip1, q1, cn1, dmat, dd, out_o):
    deg = degb[...][:, :1]
    agg2 = (_dot(agg1[...], ia1[...]) + jnp.sum(msum_p[...], axis=0)
            + deg * u2[...] + deg * de1[...])
    v2 = _dot(v1[...], ip1[...]) + _dot(agg2, q1[...]) + cn1[...]
    out_o[...] = _dot(v2, dmat[...]) + dd[...]


RB = 2000
_GRID = N_NODES // RB


def _full(shape):
    return pl.BlockSpec(shape, lambda i: (0,) * len(shape))


def _rows(width):
    return pl.BlockSpec((RB, width), lambda i: (i, 0))


def _rows2(width):
    return pl.BlockSpec((NC, RB, width), lambda i: (0, i, 0))


def _mk_node_call(body, in_specs, n_out):
    return pl.pallas_call(
        body,
        grid=(_GRID,),
        in_specs=in_specs,
        out_specs=[_rows(D_LAT)] * n_out,
        out_shape=[jax.ShapeDtypeStruct((N_NODES, D_LAT), jnp.float32)] * n_out,
    )


def kernel(node_features_in, edges_indexes, edge_features_in, params):
    src4d = edges_indexes[0].reshape(NW, NB, IB, CHUNK)
    dst4d = edges_indexes[1].reshape(NW, NB, IB, CHUNK)
    xe_ext = jnp.concatenate(
        [edge_features_in,
         jnp.ones((N_EDGES, 1), jnp.float32),
         jnp.zeros((N_EDGES, D_LAT - D_EDGE_IN - 1), jnp.float32)], axis=1)
    xe3d = xe_ext.reshape(NW * NCHUNK, CHUNK, D_LAT)
    z128 = jnp.zeros((W_ROWS, D_LAT), jnp.float32)

    p = params
    raw = []
    for mlp in (p["enc_node"], p["enc_edge"]):
        raw += [mlp[0]["W"], mlp[0]["b"].reshape(1, -1),
                mlp[1]["W"], mlp[1]["b"].reshape(1, -1)]
    for t in range(2):
        for mlp in (p["proc"][t]["edge"], p["proc"][t]["node"]):
            raw += [mlp[0]["W"], mlp[0]["b"].reshape(1, -1),
                    mlp[1]["W"], mlp[1]["b"].reshape(1, -1)]
    raw += [p["dec"][0]["W"], p["dec"][0]["b"].reshape(1, -1),
            p["dec"][1]["W"], p["dec"][1]["b"].reshape(1, -1)]

    mat = jax.ShapeDtypeStruct((D_LAT, D_LAT), jnp.float32)
    vec = jax.ShapeDtypeStruct((1, D_LAT), jnp.float32)
    prep_out = [mat, vec,
                jax.ShapeDtypeStruct((D_EDGE_IN, D_LAT), jnp.float32), vec,
                mat, mat, mat, vec, mat, mat, vec,
                mat, mat, mat, vec, mat, mat, vec,
                mat, vec]
    combos = pl.pallas_call(_tc_prep_body, out_shape=prep_out)(*raw)
    (en, bn, ee, be,
     ia0, b0, c0, de0, ip0, q0, cn0,
     ia1, b1, c1, de1, ip1, q1, cn1,
     dmat, dd) = combos

    v0, t1, u1 = _mk_node_call(
        _tc_encode_body,
        [_rows(D_LAT), _full((D_LAT, D_LAT)), _full((1, D_LAT)),
         _full((D_LAT, D_LAT)), _full((D_LAT, D_LAT))],
        3,
    )(node_features_in, en, bn, b0, c0)

    sxd_p, msum1_p = _sc_fused()(xe3d, t1, src4d, dst4d, z128)

    agg1, v1, t2, u2, degb = _mk_node_call(
        _tc_step1_body,
        [_rows2(D_LAT), _rows2(D_LAT),
         _rows(D_LAT), _rows(D_LAT),
         _full((D_EDGE_IN, D_LAT)), _full((1, D_LAT)),
         _full((D_LAT, D_LAT)), _full((1, D_LAT)),
         _full((D_LAT, D_LAT)), _full((D_LAT, D_LAT)), _full((1, D_LAT)),
         _full((D_LAT, D_LAT)), _full((D_LAT, D_LAT))],
        5,
    )(sxd_p, msum1_p, v0, u1, ee, be, ia0, de0, ip0, q0, cn0, b1, c1)

    msum2_p = _sc_spmm()(t2, src4d, dst4d, z128)

    (out,) = pl.pallas_call(
        _tc_step2_body,
        grid=(_GRID,),
        in_specs=[_rows(D_LAT), _rows(D_LAT), _rows2(D_LAT),
                  _rows(D_LAT), _rows(D_LAT),
                  _full((D_LAT, D_LAT)), _full((1, D_LAT)),
                  _full((D_LAT, D_LAT)), _full((D_LAT, D_LAT)),
                  _full((1, D_LAT)),
                  _full((D_LAT, D_LAT)), _full((1, D_LAT))],
        out_specs=[_rows(D_LAT)],
        out_shape=[jax.ShapeDtypeStruct((N_NODES, D_LAT), jnp.float32)],
    )(degb, agg1, msum2_p, v1, u2, ia1, de1, ip1, q1, cn1, dmat, dd)

    return out


# speedup vs baseline: 1.3877x; 1.3877x over previous
import functools

import jax
import jax.numpy as jnp
from jax import lax
from jax.experimental import pallas as pl
from jax.experimental.pallas import tpu as pltpu
from jax.experimental.pallas import tpu_sc as plsc

N_NODES = 10000
N_EDGES = 320000
D_LAT = 128
D_EDGE_IN = 16

NC, NS = 2, 16
NW = NC * NS
EPW = N_EDGES // NW
CHUNK = 125
NCHUNK = EPW // CHUNK
IB = 20
NB = NCHUNK // IB
W_STEP = 624
W_ROWS = 640

_HIGH = jax.lax.Precision.HIGHEST


def _dot(a, b):
    return jnp.dot(a, b, precision=_HIGH, preferred_element_type=jnp.float32)


def _acc_window(ref, s):
    return ref.at[pl.ds(s * W_STEP, W_ROWS)]


def _spmm_phase(t_hbm, src4d_hbm, dst4d_hbm,
                src_v, dst_v, rows_a, rows_b, acc, sem_a, sem_b, wid):
    def outer(b, carry):
        pltpu.sync_copy(src4d_hbm.at[wid, b], src_v)
        pltpu.sync_copy(dst4d_hbm.at[wid, b], dst_v)
        pltpu.async_copy(t_hbm.at[src_v.at[0]], rows_a, sem_a)

        def body(jj, carry2):
            @pl.when(jj % 2 == 0)
            def _even():
                @pl.when(jj + 1 < IB)
                def _():
                    pltpu.async_copy(t_hbm.at[src_v.at[jj + 1]], rows_b, sem_b)
                pltpu.make_async_copy(t_hbm.at[src_v.at[jj]], rows_a,
                                      sem_a).wait()
                pltpu.sync_copy(rows_a, acc.at[dst_v.at[jj]], add=True)

            @pl.when(jj % 2 == 1)
            def _odd():
                @pl.when(jj + 1 < IB)
                def _():
                    pltpu.async_copy(t_hbm.at[src_v.at[jj + 1]], rows_a, sem_a)
                pltpu.make_async_copy(t_hbm.at[src_v.at[jj]], rows_b,
                                      sem_b).wait()
                pltpu.sync_copy(rows_b, acc.at[dst_v.at[jj]], add=True)

            return carry2

        return lax.fori_loop(0, IB, body, carry)

    lax.fori_loop(0, NB, outer, 0)


def _segsum_phase(xe_hbm, dst4d_hbm, dst_v, rows_a, rows_b, acc,
                  sem_a, sem_b, wid):
    def outer(b, carry):
        pltpu.sync_copy(dst4d_hbm.at[wid, b], dst_v)
        pltpu.async_copy(xe_hbm.at[wid * NCHUNK + b * IB], rows_a, sem_a)

        def body(jj, carry2):
            j = b * IB + jj

            @pl.when(jj % 2 == 0)
            def _even():
                @pl.when(jj + 1 < IB)
                def _():
                    pltpu.async_copy(xe_hbm.at[wid * NCHUNK + j + 1], rows_b,
                                     sem_b)
                pltpu.make_async_copy(xe_hbm.at[wid * NCHUNK + j], rows_a,
                                      sem_a).wait()
                pltpu.sync_copy(rows_a, acc.at[dst_v.at[jj]], add=True)

            @pl.when(jj % 2 == 1)
            def _odd():
                @pl.when(jj + 1 < IB)
                def _():
                    pltpu.async_copy(xe_hbm.at[wid * NCHUNK + j + 1], rows_a,
                                     sem_a)
                pltpu.make_async_copy(xe_hbm.at[wid * NCHUNK + j], rows_b,
                                      sem_b).wait()
                pltpu.sync_copy(rows_b, acc.at[dst_v.at[jj]], add=True)

            return carry2

        return lax.fori_loop(0, IB, body, carry)

    lax.fori_loop(0, NB, outer, 0)


def _sc_fused_body(xe_hbm, t_hbm, src4d_hbm, dst4d_hbm, z_hbm,
                   sxd_out, msum_out,
                   src_v, dst_v, rows_a, rows_b, acc, sem_a, sem_b):
    c = lax.axis_index("c")
    s = lax.axis_index("s")
    wid = s * NC + c

    pltpu.sync_copy(z_hbm, _acc_window(acc, s))
    plsc.subcore_barrier()
    _segsum_phase(xe_hbm, dst4d_hbm, dst_v, rows_a, rows_b, acc,
                  sem_a, sem_b, wid)
    plsc.subcore_barrier()
    pltpu.sync_copy(_acc_window(acc, s), _acc_window(sxd_out.at[c], s))
    plsc.subcore_barrier()
    pltpu.sync_copy(z_hbm, _acc_window(acc, s))
    plsc.subcore_barrier()
    _spmm_phase(t_hbm, src4d_hbm, dst4d_hbm,
                src_v, dst_v, rows_a, rows_b, acc, sem_a, sem_b, wid)
    plsc.subcore_barrier()
    pltpu.sync_copy(_acc_window(acc, s), _acc_window(msum_out.at[c], s))


def _sc_spmm_body(t_hbm, src4d_hbm, dst4d_hbm, z_hbm,
                  msum_out,
                  src_v, dst_v, rows_a, rows_b, acc, sem_a, sem_b):
    c = lax.axis_index("c")
    s = lax.axis_index("s")
    wid = s * NC + c

    pltpu.sync_copy(z_hbm, _acc_window(acc, s))
    plsc.subcore_barrier()
    _spmm_phase(t_hbm, src4d_hbm, dst4d_hbm,
                src_v, dst_v, rows_a, rows_b, acc, sem_a, sem_b, wid)
    plsc.subcore_barrier()
    pltpu.sync_copy(_acc_window(acc, s), _acc_window(msum_out.at[c], s))


_SC_SCRATCH = [
    pltpu.VMEM((IB, CHUNK), jnp.int32),
    pltpu.VMEM((IB, CHUNK), jnp.int32),
    pltpu.VMEM((CHUNK, D_LAT), jnp.float32),
    pltpu.VMEM((CHUNK, D_LAT), jnp.float32),
    pltpu.VMEM_SHARED((N_NODES, D_LAT), jnp.float32),
    pltpu.SemaphoreType.DMA,
    pltpu.SemaphoreType.DMA,
]
_SC_MESH = dict(core_axis_name="c", subcore_axis_name="s",
                num_cores=NC, num_subcores=NS)
_OUT128 = jax.ShapeDtypeStruct((NC, N_NODES, D_LAT), jnp.float32)


@functools.cache
def _sc_fused():
    return pl.kernel(
        _sc_fused_body,
        out_type=(_OUT128, _OUT128),
        mesh=plsc.VectorSubcoreMesh(**_SC_MESH),
        scratch_types=list(_SC_SCRATCH),
    )


@functools.cache
def _sc_spmm():
    return pl.kernel(
        _sc_spmm_body,
        out_type=_OUT128,
        mesh=plsc.VectorSubcoreMesh(**_SC_MESH),
        scratch_types=list(_SC_SCRATCH),
    )


def _tc_prep_body(wn1, bn1, wn2, bn2, we1, be1, we2, be2,
                  ew1_0, eb1_0, ew2_0, eb2_0, nw1_0, nb1_0, nw2_0, nb2_0,
                  ew1_1, eb1_1, ew2_1, eb2_1, nw1_1, nb1_1, nw2_1, nb2_1,
                  dw1, db1, dw2, db2,
                  en_o, bn_o, ee_o, be_o,
                  ia0_o, b0_o, c0_o, de0_o, ip0_o, q0_o, cn0_o,
                  ia1_o, b1_o, c1_o, de1_o, ip1_o, q1_o, cn1_o,
                  d_o, dd_o):
    eye = jnp.eye(D_LAT, dtype=jnp.float32)
    en_o[...] = _dot(wn1[...], wn2[...])
    bn_o[...] = _dot(bn1[...], wn2[...]) + bn2[...]
    ee_o[...] = _dot(we1[...], we2[...])
    be_o[...] = _dot(be1[...], we2[...]) + be2[...]

    for (ew1, eb1, ew2, eb2, nw1, nb1, nw2, nb2,
         ia_o, b_o, c_o, de_o, ip_o, q_o, cn_o) in (
            (ew1_0, eb1_0, ew2_0, eb2_0, nw1_0, nb1_0, nw2_0, nb2_0,
             ia0_o, b0_o, c0_o, de0_o, ip0_o, q0_o, cn0_o),
            (ew1_1, eb1_1, ew2_1, eb2_1, nw1_1, nb1_1, nw2_1, nb2_1,
             ia1_o, b1_o, c1_o, de1_o, ip1_o, q1_o, cn1_o)):
        w1 = ew1[...]
        w2 = ew2[...]
        ia_o[...] = eye + _dot(w1[:D_LAT], w2)
        b_o[...] = _dot(w1[D_LAT:2 * D_LAT], w2)
        c_o[...] = _dot(w1[2 * D_LAT:], w2)
        de_o[...] = _dot(eb1[...], w2) + eb2[...]
        m1 = nw1[...]
        m2 = nw2[...]
        ip_o[...] = eye + _dot(m1[:D_LAT], m2)
        q_o[...] = _dot(m1[D_LAT:], m2)
        cn_o[...] = _dot(nb1[...], m2) + nb2[...]

    d_o[...] = _dot(dw1[...], dw2[...])
    dd_o[...] = _dot(db1[...], dw2[...]) + db2[...]


def _tc_encode_body(xn, en, bn, b0, c0, v0_o, t1_o, u1_o):
    v0 = _dot(xn[...], en[...]) + bn[...]
    v0_o[...] = v0
    t1_o[...] = _dot(v0, b0[...])
    u1_o[...] = _dot(v0, c0[...])


def _tc_step1_body(sxd_p, msum_p, v0, u1,
                   ee, be, ia0, de0, ip0, q0, cn0, b1, c1,
                   agg1_o, v1_o, t2_o, u2_o, deg_o):
    sxd = jnp.sum(sxd_p[...], axis=0)
    msum_a = msum_p[...]
    sx = sxd[:, :D_EDGE_IN]
    deg = sxd[:, D_EDGE_IN:D_EDGE_IN + 1]
    deg_o[...] = jnp.broadcast_to(deg, deg_o.shape)
    eagg0 = _dot(sx, ee[...]) + deg * be[...]
    agg1 = (_dot(eagg0, ia0[...]) + jnp.sum(msum_a, axis=0)
            + deg * u1[...] + deg * de0[...])
    v1 = _dot(v0[...], ip0[...]) + _dot(agg1, q0[...]) + cn0[...]
    agg1_o[...] = agg1
    v1_o[...] = v1
    t2_o[...] = _dot(v1, b1[...])
    u2_o[...] = _dot(v1, c1[...])


def _tc_step2_body(degb, agg1, msum_p, v1, u2,
                   ia1, de1, ip1, q1, cn1, dmat, dd, out_o):
    deg = degb[...][:, :1]
    agg2 = (_dot(agg1[...], ia1[...]) + jnp.sum(msum_p[...], axis=0)
            + deg * u2[...] + deg * de1[...])
    v2 = _dot(v1[...], ip1[...]) + _dot(agg2, q1[...]) + cn1[...]
    out_o[...] = _dot(v2, dmat[...]) + dd[...]


RB = 2000
_GRID = N_NODES // RB


def _full(shape):
    return pl.BlockSpec(shape, lambda i: (0,) * len(shape))


def _rows(width):
    return pl.BlockSpec((RB, width), lambda i: (i, 0))


def _rows2(width):
    return pl.BlockSpec((NC, RB, width), lambda i: (0, i, 0))


def _mk_node_call(body, in_specs, n_out):
    return pl.pallas_call(
        body,
        grid=(_GRID,),
        in_specs=in_specs,
        out_specs=[_rows(D_LAT)] * n_out,
        out_shape=[jax.ShapeDtypeStruct((N_NODES, D_LAT), jnp.float32)] * n_out,
    )


def kernel(node_features_in, edges_indexes, edge_features_in, params):
    src4d = edges_indexes[0].reshape(NW, NB, IB, CHUNK)
    dst4d = edges_indexes[1].reshape(NW, NB, IB, CHUNK)
    xe_ext = jnp.concatenate(
        [edge_features_in,
         jnp.ones((N_EDGES, 1), jnp.float32),
         jnp.zeros((N_EDGES, D_LAT - D_EDGE_IN - 1), jnp.float32)], axis=1)
    xe3d = xe_ext.reshape(NW * NCHUNK, CHUNK, D_LAT)
    z128 = jnp.zeros((W_ROWS, D_LAT), jnp.float32)

    p = params
    raw = []
    for mlp in (p["enc_node"], p["enc_edge"]):
        raw += [mlp[0]["W"], mlp[0]["b"].reshape(1, -1),
                mlp[1]["W"], mlp[1]["b"].reshape(1, -1)]
    for t in range(2):
        for mlp in (p["proc"][t]["edge"], p["proc"][t]["node"]):
            raw += [mlp[0]["W"], mlp[0]["b"].reshape(1, -1),
                    mlp[1]["W"], mlp[1]["b"].reshape(1, -1)]
    raw += [p["dec"][0]["W"], p["dec"][0]["b"].reshape(1, -1),
            p["dec"][1]["W"], p["dec"][1]["b"].reshape(1, -1)]

    mat = jax.ShapeDtypeStruct((D_LAT, D_LAT), jnp.float32)
    vec = jax.ShapeDtypeStruct((1, D_LAT), jnp.float32)
    prep_out = [mat, vec,
                jax.ShapeDtypeStruct((D_EDGE_IN, D_LAT), jnp.float32), vec,
                mat, mat, mat, vec, mat, mat, vec,
                mat, mat, mat, vec, mat, mat, vec,
                mat, vec]
    combos = pl.pallas_call(_tc_prep_body, out_shape=prep_out)(*raw)
    (en, bn, ee, be,
     ia0, b0, c0, de0, ip0, q0, cn0,
     ia1, b1, c1, de1, ip1, q1, cn1,
     dmat, dd) = combos

    v0, t1, u1 = _mk_node_call(
        _tc_encode_body,
        [_rows(D_LAT), _full((D_LAT, D_LAT)), _full((1, D_LAT)),
         _full((D_LAT, D_LAT)), _full((D_LAT, D_LAT))],
        3,
    )(node_features_in, en, bn, b0, c0)

    sxd_p, msum1_p = _sc_fused()(xe3d, t1, src4d, dst4d, z128)

    agg1, v1, t2, u2, degb = _mk_node_call(
        _tc_step1_body,
        [_rows2(D_LAT), _rows2(D_LAT),
         _rows(D_LAT), _rows(D_LAT),
         _full((D_EDGE_IN, D_LAT)), _full((1, D_LAT)),
         _full((D_LAT, D_LAT)), _full((1, D_LAT)),
         _full((D_LAT, D_LAT)), _full((D_LAT, D_LAT)), _full((1, D_LAT)),
         _full((D_LAT, D_LAT)), _full((D_LAT, D_LAT))],
        5,
    )(sxd_p, msum1_p, v0, u1, ee, be, ia0, de0, ip0, q0, cn0, b1, c1)

    msum2_p = _sc_spmm()(t2, src4d, dst4d, z128)

    (out,) = pl.pallas_call(
        _tc_step2_body,
        grid=(_GRID,),
        in_specs=[_rows(D_LAT), _rows(D_LAT), _rows2(D_LAT),
                  _rows(D_LAT), _rows(D_LAT),
                  _full((D_LAT, D_LAT)), _full((1, D_LAT)),
                  _full((D_LAT, D_LAT)), _full((D_LAT, D_LAT)),
                  _full((1, D_LAT)),
                  _full((D_LAT, D_LAT)), _full((1, D_LAT))],
        out_specs=[_rows(D_LAT)],
        out_shape=[jax.ShapeDtypeStruct((N_NODES, D_LAT), jnp.float32)],
    )(degb, agg1, msum2_p, v1, u2, ia1, de1, ip1, q1, cn1, dmat, dd)

    return out


# speedup vs baseline: 1.4164x; 1.0207x over previous
import functools

import jax
import jax.numpy as jnp
from jax import lax
from jax.experimental import pallas as pl
from jax.experimental.pallas import tpu as pltpu
from jax.experimental.pallas import tpu_sc as plsc

N_NODES = 10000
N_EDGES = 320000
D_LAT = 128
D_EDGE_IN = 16

NC, NS = 2, 16
NW = NC * NS
EPW = N_EDGES // NW
CHUNK = 125
NCHUNK = EPW // CHUNK
IB = 40
NB = NCHUNK // IB
W_STEP = 624
W_ROWS = 640

_HIGH = jax.lax.Precision.HIGHEST


def _dot(a, b):
    return jnp.dot(a, b, precision=_HIGH, preferred_element_type=jnp.float32)


def _acc_window(ref, s):
    return ref.at[pl.ds(s * W_STEP, W_ROWS)]


def _spmm_phase(t_hbm, src4d_hbm, dst4d_hbm,
                src_v, dst_v, rows_a, rows_b, acc, sem_a, sem_b, wid):
    def outer(b, carry):
        pltpu.sync_copy(src4d_hbm.at[wid, b], src_v)
        pltpu.sync_copy(dst4d_hbm.at[wid, b], dst_v)
        pltpu.async_copy(t_hbm.at[src_v.at[0]], rows_a, sem_a)

        def body(jj, carry2):
            @pl.when(jj % 2 == 0)
            def _even():
                @pl.when(jj + 1 < IB)
                def _():
                    pltpu.async_copy(t_hbm.at[src_v.at[jj + 1]], rows_b, sem_b)
                pltpu.make_async_copy(t_hbm.at[src_v.at[jj]], rows_a,
                                      sem_a).wait()
                pltpu.sync_copy(rows_a, acc.at[dst_v.at[jj]], add=True)

            @pl.when(jj % 2 == 1)
            def _odd():
                @pl.when(jj + 1 < IB)
                def _():
                    pltpu.async_copy(t_hbm.at[src_v.at[jj + 1]], rows_a, sem_a)
                pltpu.make_async_copy(t_hbm.at[src_v.at[jj]], rows_b,
                                      sem_b).wait()
                pltpu.sync_copy(rows_b, acc.at[dst_v.at[jj]], add=True)

            return carry2

        return lax.fori_loop(0, IB, body, carry)

    lax.fori_loop(0, NB, outer, 0)


def _segsum_phase(xe_hbm, dst4d_hbm, dst_v, rows_a, rows_b, acc,
                  sem_a, sem_b, wid):
    def outer(b, carry):
        pltpu.sync_copy(dst4d_hbm.at[wid, b], dst_v)
        pltpu.async_copy(xe_hbm.at[wid * NCHUNK + b * IB], rows_a, sem_a)

        def body(jj, carry2):
            j = b * IB + jj

            @pl.when(jj % 2 == 0)
            def _even():
                @pl.when(jj + 1 < IB)
                def _():
                    pltpu.async_copy(xe_hbm.at[wid * NCHUNK + j + 1], rows_b,
                                     sem_b)
                pltpu.make_async_copy(xe_hbm.at[wid * NCHUNK + j], rows_a,
                                      sem_a).wait()
                pltpu.sync_copy(rows_a, acc.at[dst_v.at[jj]], add=True)

            @pl.when(jj % 2 == 1)
            def _odd():
                @pl.when(jj + 1 < IB)
                def _():
                    pltpu.async_copy(xe_hbm.at[wid * NCHUNK + j + 1], rows_a,
                                     sem_a)
                pltpu.make_async_copy(xe_hbm.at[wid * NCHUNK + j], rows_b,
                                      sem_b).wait()
                pltpu.sync_copy(rows_b, acc.at[dst_v.at[jj]], add=True)

            return carry2

        return lax.fori_loop(0, IB, body, carry)

    lax.fori_loop(0, NB, outer, 0)


def _sc_fused_body(xe_hbm, t_hbm, src4d_hbm, dst4d_hbm, z_hbm,
                   sxd_out, msum_out,
                   src_v, dst_v, rows_a, rows_b, acc, sem_a, sem_b):
    c = lax.axis_index("c")
    s = lax.axis_index("s")
    wid = s * NC + c

    pltpu.sync_copy(z_hbm, _acc_window(acc, s))
    plsc.subcore_barrier()
    _segsum_phase(xe_hbm, dst4d_hbm, dst_v, rows_a, rows_b, acc,
                  sem_a, sem_b, wid)
    plsc.subcore_barrier()
    pltpu.sync_copy(_acc_window(acc, s), _acc_window(sxd_out.at[c], s))
    plsc.subcore_barrier()
    pltpu.sync_copy(z_hbm, _acc_window(acc, s))
    plsc.subcore_barrier()
    _spmm_phase(t_hbm, src4d_hbm, dst4d_hbm,
                src_v, dst_v, rows_a, rows_b, acc, sem_a, sem_b, wid)
    plsc.subcore_barrier()
    pltpu.sync_copy(_acc_window(acc, s), _acc_window(msum_out.at[c], s))


def _sc_spmm_body(t_hbm, src4d_hbm, dst4d_hbm, z_hbm,
                  msum_out,
                  src_v, dst_v, rows_a, rows_b, acc, sem_a, sem_b):
    c = lax.axis_index("c")
    s = lax.axis_index("s")
    wid = s * NC + c

    pltpu.sync_copy(z_hbm, _acc_window(acc, s))
    plsc.subcore_barrier()
    _spmm_phase(t_hbm, src4d_hbm, dst4d_hbm,
                src_v, dst_v, rows_a, rows_b, acc, sem_a, sem_b, wid)
    plsc.subcore_barrier()
    pltpu.sync_copy(_acc_window(acc, s), _acc_window(msum_out.at[c], s))


_SC_SCRATCH = [
    pltpu.VMEM((IB, CHUNK), jnp.int32),
    pltpu.VMEM((IB, CHUNK), jnp.int32),
    pltpu.VMEM((CHUNK, D_LAT), jnp.float32),
    pltpu.VMEM((CHUNK, D_LAT), jnp.float32),
    pltpu.VMEM_SHARED((N_NODES, D_LAT), jnp.float32),
    pltpu.SemaphoreType.DMA,
    pltpu.SemaphoreType.DMA,
]
_SC_MESH = dict(core_axis_name="c", subcore_axis_name="s",
                num_cores=NC, num_subcores=NS)
_OUT128 = jax.ShapeDtypeStruct((NC, N_NODES, D_LAT), jnp.float32)


@functools.cache
def _sc_fused():
    return pl.kernel(
        _sc_fused_body,
        out_type=(_OUT128, _OUT128),
        mesh=plsc.VectorSubcoreMesh(**_SC_MESH),
        scratch_types=list(_SC_SCRATCH),
    )


@functools.cache
def _sc_spmm():
    return pl.kernel(
        _sc_spmm_body,
        out_type=_OUT128,
        mesh=plsc.VectorSubcoreMesh(**_SC_MESH),
        scratch_types=list(_SC_SCRATCH),
    )


def _tc_prep_body(wn1, bn1, wn2, bn2, we1, be1, we2, be2,
                  ew1_0, eb1_0, ew2_0, eb2_0, nw1_0, nb1_0, nw2_0, nb2_0,
                  ew1_1, eb1_1, ew2_1, eb2_1, nw1_1, nb1_1, nw2_1, nb2_1,
                  dw1, db1, dw2, db2,
                  en_o, bn_o, ee_o, be_o,
                  ia0_o, b0_o, c0_o, de0_o, ip0_o, q0_o, cn0_o,
                  ia1_o, b1_o, c1_o, de1_o, ip1_o, q1_o, cn1_o,
                  d_o, dd_o):
    eye = jnp.eye(D_LAT, dtype=jnp.float32)
    en_o[...] = _dot(wn1[...], wn2[...])
    bn_o[...] = _dot(bn1[...], wn2[...]) + bn2[...]
    ee_o[...] = _dot(we1[...], we2[...])
    be_o[...] = _dot(be1[...], we2[...]) + be2[...]

    for (ew1, eb1, ew2, eb2, nw1, nb1, nw2, nb2,
         ia_o, b_o, c_o, de_o, ip_o, q_o, cn_o) in (
            (ew1_0, eb1_0, ew2_0, eb2_0, nw1_0, nb1_0, nw2_0, nb2_0,
             ia0_o, b0_o, c0_o, de0_o, ip0_o, q0_o, cn0_o),
            (ew1_1, eb1_1, ew2_1, eb2_1, nw1_1, nb1_1, nw2_1, nb2_1,
             ia1_o, b1_o, c1_o, de1_o, ip1_o, q1_o, cn1_o)):
        w1 = ew1[...]
        w2 = ew2[...]
        ia_o[...] = eye + _dot(w1[:D_LAT], w2)
        b_o[...] = _dot(w1[D_LAT:2 * D_LAT], w2)
        c_o[...] = _dot(w1[2 * D_LAT:], w2)
        de_o[...] = _dot(eb1[...], w2) + eb2[...]
        m1 = nw1[...]
        m2 = nw2[...]
        ip_o[...] = eye + _dot(m1[:D_LAT], m2)
        q_o[...] = _dot(m1[D_LAT:], m2)
        cn_o[...] = _dot(nb1[...], m2) + nb2[...]

    d_o[...] = _dot(dw1[...], dw2[...])
    dd_o[...] = _dot(db1[...], dw2[...]) + db2[...]


def _tc_encode_body(xn, en, bn, b0, c0, v0_o, t1_o, u1_o):
    v0 = _dot(xn[...], en[...]) + bn[...]
    v0_o[...] = v0
    t1_o[...] = _dot(v0, b0[...])
    u1_o[...] = _dot(v0, c0[...])


def _tc_step1_body(sxd_p, msum_p, v0, u1,
                   ee, be, ia0, de0, ip0, q0, cn0, b1, c1,
                   agg1_o, v1_o, t2_o, u2_o, deg_o):
    sxd = jnp.sum(sxd_p[...], axis=0)
    msum_a = msum_p[...]
    sx = sxd[:, :D_EDGE_IN]
    deg = sxd[:, D_EDGE_IN:D_EDGE_IN + 1]
    deg_o[...] = jnp.broadcast_to(deg, deg_o.shape)
    eagg0 = _dot(sx, ee[...]) + deg * be[...]
    agg1 = (_dot(eagg0, ia0[...]) + jnp.sum(msum_a, axis=0)
            + deg * u1[...] + deg * de0[...])
    v1 = _dot(v0[...], ip0[...]) + _dot(agg1, q0[...]) + cn0[...]
    agg1_o[...] = agg1
    v1_o[...] = v1
    t2_o[...] = _dot(v1, b1[...])
    u2_o[...] = _dot(v1, c1[...])


def _tc_step2_body(degb, agg1, msum_p, v1, u2,
                   ia1, de1, ip1, q1, cn1, dmat, dd, out_o):
    deg = degb[...][:, :1]
    agg2 = (_dot(agg1[...], ia1[...]) + jnp.sum(msum_p[...], axis=0)
            + deg * u2[...] + deg * de1[...])
    v2 = _dot(v1[...], ip1[...]) + _dot(agg2, q1[...]) + cn1[...]
    out_o[...] = _dot(v2, dmat[...]) + dd[...]


RB = 2000
_GRID = N_NODES // RB


def _full(shape):
    return pl.BlockSpec(shape, lambda i: (0,) * len(shape))


def _rows(width):
    return pl.BlockSpec((RB, width), lambda i: (i, 0))


def _rows2(width):
    return pl.BlockSpec((NC, RB, width), lambda i: (0, i, 0))


def _mk_node_call(body, in_specs, n_out):
    return pl.pallas_call(
        body,
        grid=(_GRID,),
        in_specs=in_specs,
        out_specs=[_rows(D_LAT)] * n_out,
        out_shape=[jax.ShapeDtypeStruct((N_NODES, D_LAT), jnp.float32)] * n_out,
    )


def kernel(node_features_in, edges_indexes, edge_features_in, params):
    src4d = edges_indexes[0].reshape(NW, NB, IB, CHUNK)
    dst4d = edges_indexes[1].reshape(NW, NB, IB, CHUNK)
    xe_ext = jnp.concatenate(
        [edge_features_in,
         jnp.ones((N_EDGES, 1), jnp.float32),
         jnp.zeros((N_EDGES, D_LAT - D_EDGE_IN - 1), jnp.float32)], axis=1)
    xe3d = xe_ext.reshape(NW * NCHUNK, CHUNK, D_LAT)
    z128 = jnp.zeros((W_ROWS, D_LAT), jnp.float32)

    p = params
    raw = []
    for mlp in (p["enc_node"], p["enc_edge"]):
        raw += [mlp[0]["W"], mlp[0]["b"].reshape(1, -1),
                mlp[1]["W"], mlp[1]["b"].reshape(1, -1)]
    for t in range(2):
        for mlp in (p["proc"][t]["edge"], p["proc"][t]["node"]):
            raw += [mlp[0]["W"], mlp[0]["b"].reshape(1, -1),
                    mlp[1]["W"], mlp[1]["b"].reshape(1, -1)]
    raw += [p["dec"][0]["W"], p["dec"][0]["b"].reshape(1, -1),
            p["dec"][1]["W"], p["dec"][1]["b"].reshape(1, -1)]

    mat = jax.ShapeDtypeStruct((D_LAT, D_LAT), jnp.float32)
    vec = jax.ShapeDtypeStruct((1, D_LAT), jnp.float32)
    prep_out = [mat, vec,
                jax.ShapeDtypeStruct((D_EDGE_IN, D_LAT), jnp.float32), vec,
                mat, mat, mat, vec, mat, mat, vec,
                mat, mat, mat, vec, mat, mat, vec,
                mat, vec]
    combos = pl.pallas_call(_tc_prep_body, out_shape=prep_out)(*raw)
    (en, bn, ee, be,
     ia0, b0, c0, de0, ip0, q0, cn0,
     ia1, b1, c1, de1, ip1, q1, cn1,
     dmat, dd) = combos

    v0, t1, u1 = _mk_node_call(
        _tc_encode_body,
        [_rows(D_LAT), _full((D_LAT, D_LAT)), _full((1, D_LAT)),
         _full((D_LAT, D_LAT)), _full((D_LAT, D_LAT))],
        3,
    )(node_features_in, en, bn, b0, c0)

    sxd_p, msum1_p = _sc_fused()(xe3d, t1, src4d, dst4d, z128)

    agg1, v1, t2, u2, degb = _mk_node_call(
        _tc_step1_body,
        [_rows2(D_LAT), _rows2(D_LAT),
         _rows(D_LAT), _rows(D_LAT),
         _full((D_EDGE_IN, D_LAT)), _full((1, D_LAT)),
         _full((D_LAT, D_LAT)), _full((1, D_LAT)),
         _full((D_LAT, D_LAT)), _full((D_LAT, D_LAT)), _full((1, D_LAT)),
         _full((D_LAT, D_LAT)), _full((D_LAT, D_LAT))],
        5,
    )(sxd_p, msum1_p, v0, u1, ee, be, ia0, de0, ip0, q0, cn0, b1, c1)

    msum2_p = _sc_spmm()(t2, src4d, dst4d, z128)

    (out,) = pl.pallas_call(
        _tc_step2_body,
        grid=(_GRID,),
        in_specs=[_rows(D_LAT), _rows(D_LAT), _rows2(D_LAT),
                  _rows(D_LAT), _rows(D_LAT),
                  _full((D_LAT, D_LAT)), _full((1, D_LAT)),
                  _full((D_LAT, D_LAT)), _full((D_LAT, D_LAT)),
                  _full((1, D_LAT)),
                  _full((D_LAT, D_LAT)), _full((1, D_LAT))],
        out_specs=[_rows(D_LAT)],
        out_shape=[jax.ShapeDtypeStruct((N_NODES, D_LAT), jnp.float32)],
    )(degb, agg1, msum2_p, v1, u2, ia1, de1, ip1, q1, cn1, dmat, dd)

    return out
